# baseline (speedup 1.0000x reference)
import functools

import jax
import jax.numpy as jnp
from jax import lax
from jax.experimental import pallas as pl
from jax.experimental.pallas import tpu as pltpu
from jax.experimental.pallas import tpu_sc as plsc

N = 10000
NPAD = 10240
E = 320000
H = 20
HH = 10
HD = 16
NC = 2
NS = 16
NW = NC * NS
EPW = E // NW
EPT = E // NS
EB = 1000
NBLK2 = EPT // EB
RPS = NPAD // NS
PR = NPAD // 8
PRN = N // 8
DR = NPAD // 128

_sc_mesh = plsc.VectorSubcoreMesh(core_axis_name="c", subcore_axis_name="s")


NHB = NBLK2 // 2


@functools.partial(
    pl.kernel,
    out_type=jax.ShapeDtypeStruct((NC, 2, NPAD), jnp.float32),
    mesh=_sc_mesh,
    scratch_types=[
        pltpu.VMEM_SHARED((NPAD,), jnp.float32),
        pltpu.VMEM_SHARED((NPAD,), jnp.float32),
        pltpu.VMEM((NHB, EB), jnp.int32),
        pltpu.VMEM((NHB, EB), jnp.int32),
        pltpu.VMEM((EB,), jnp.float32),
    ],
    compiler_params=pltpu.CompilerParams(use_tc_tiling_on_sc=False),
)
def _degrees_sc(ei4_hbm, ones_hbm, zeros_hbm, out_hbm,
                acc_o, acc_i, idx_s, idx_d, ones_v):
    cid = lax.axis_index("c")
    sid = lax.axis_index("s")
    wid = cid * NS + sid

    row0 = sid * RPS
    pltpu.sync_copy(zeros_hbm.at[0, pl.ds(row0, RPS)], acc_o.at[pl.ds(row0, RPS)])
    pltpu.sync_copy(zeros_hbm.at[1, pl.ds(row0, RPS)], acc_i.at[pl.ds(row0, RPS)])
    pltpu.sync_copy(ei4_hbm.at[0, wid], idx_s)
    pltpu.sync_copy(ei4_hbm.at[1, wid], idx_d)
    pltpu.sync_copy(ones_hbm, ones_v)
    plsc.subcore_barrier()

    def body(r, carry):
        pltpu.sync_copy(ones_v, acc_o.at[idx_s.at[r]], add=True)
        pltpu.sync_copy(ones_v, acc_i.at[idx_d.at[r]], add=True)
        return carry
    lax.fori_loop(0, NHB, body, 0)

    plsc.subcore_barrier()
    pltpu.sync_copy(acc_o.at[pl.ds(row0, RPS)], out_hbm.at[cid, 0, pl.ds(row0, RPS)])
    pltpu.sync_copy(acc_i.at[pl.ds(row0, RPS)], out_hbm.at[cid, 1, pl.ds(row0, RPS)])


@functools.partial(
    pl.kernel,
    out_type=(
        jax.ShapeDtypeStruct((NC, NPAD, HD), jnp.float32),
        jax.ShapeDtypeStruct((NC, NPAD, HD), jnp.float32),
        jax.ShapeDtypeStruct((NC, NPAD, HD), jnp.float32),
    ),
    mesh=_sc_mesh,
    scratch_types=[
        pltpu.VMEM_SHARED((NPAD, HD), jnp.float32),
        pltpu.VMEM_SHARED((NPAD, HD), jnp.float32),
        pltpu.VMEM((NBLK2, EB), jnp.int32),
        pltpu.VMEM((NBLK2, EB), jnp.int32),
        pltpu.VMEM((EB, HD), jnp.float32),
        pltpu.VMEM((EB, HD), jnp.float32),
        pltpu.VMEM((RPS, HD), jnp.float32),
        pltpu.VMEM((RPS, HD), jnp.float32),
        pltpu.SemaphoreType.DMA,
        pltpu.SemaphoreType.DMA,
    ],
    compiler_params=pltpu.CompilerParams(use_tc_tiling_on_sc=False),
)
def _round_sc(feat0_hbm, feat1_hbm, ei4_hbm, cbrd_hbm, zeros16_hbm,
              s1_out, s2_out, featb_out,
              accA, accB, srcv, dstv, rows0, rows1, bufA, bufC,
              sem0, sem1):
    cid = lax.axis_index("c")
    sid = lax.axis_index("s")
    row0 = sid * RPS

    pltpu.sync_copy(zeros16_hbm.at[pl.ds(row0, RPS)], accA.at[pl.ds(row0, RPS)])
    pltpu.sync_copy(zeros16_hbm.at[pl.ds(row0, RPS)], accB.at[pl.ds(row0, RPS)])
    pltpu.sync_copy(ei4_hbm.at[0, sid], srcv)
    pltpu.sync_copy(ei4_hbm.at[1, sid], dstv)
    plsc.subcore_barrier()

    def _edge_pass(gsrc, acc):
        pltpu.async_copy(gsrc.at[srcv.at[0]], rows0, sem0)

        def body(i, carry):
            b0 = 2 * i
            pltpu.async_copy(gsrc.at[srcv.at[b0 + 1]], rows1, sem1)
            pltpu.make_async_copy(gsrc.at[srcv.at[b0]], rows0, sem0).wait()
            pltpu.sync_copy(rows0, acc.at[dstv.at[b0]], add=True)

            @pl.when(i < NBLK2 // 2 - 1)
            def _():
                pltpu.async_copy(gsrc.at[srcv.at[b0 + 2]], rows0, sem0)

            pltpu.make_async_copy(gsrc.at[srcv.at[b0 + 1]], rows1, sem1).wait()
            pltpu.sync_copy(rows1, acc.at[dstv.at[b0 + 1]], add=True)
            return carry
        lax.fori_loop(0, NBLK2 // 2, body, 0)

    @pl.when(cid == 0)
    def _():
        _edge_pass(feat0_hbm, accA)

    @pl.when(cid == 1)
    def _():
        _edge_pass(feat1_hbm, accA)

    plsc.subcore_barrier()
    pltpu.sync_copy(accA.at[pl.ds(row0, RPS)], s1_out.at[cid, pl.ds(row0, RPS)])

    pltpu.sync_copy(accA.at[pl.ds(row0, RPS)], bufA)
    pltpu.sync_copy(cbrd_hbm.at[pl.ds(row0, RPS)], bufC)

    def scale(j, carry):
        bufA[j, :] = bufA[j, :] * bufC[j, :]
        return carry
    lax.fori_loop(0, RPS, scale, 0)
    pltpu.sync_copy(bufA, featb_out.at[cid, pl.ds(row0, RPS)])
    plsc.subcore_barrier()

    _edge_pass(featb_out.at[cid], accB)

    plsc.subcore_barrier()
    pltpu.sync_copy(accB.at[pl.ds(row0, RPS)], s2_out.at[cid, pl.ds(row0, RPS)])


def _mm(a, b):
    return jnp.dot(a, b, preferred_element_type=jnp.float32)


def _prep_body(x8_ref, wa_ref, wb_ref, bba_ref, bbb_ref, degp_ref, p16_ref,
               h0a_ref, h0b_ref, g1a_ref, g1b_ref, ap_ref, bp_ref, cpp_ref):
    x8 = x8_ref[...]
    zr = jnp.zeros((PR - PRN, 128), jnp.float32)
    h0a = jnp.concatenate(
        [jnp.maximum(_mm(x8, wa_ref[...]) + bba_ref[...], 0.0), zr], 0)
    h0b = jnp.concatenate(
        [jnp.maximum(_mm(x8, wb_ref[...]) + bbb_ref[...], 0.0), zr], 0)
    deg = degp_ref[0] + degp_ref[1]
    av = jnp.where(deg[0] > 0.5, lax.rsqrt(jnp.maximum(deg[0], 1.0)), 0.0)
    bv = jnp.where(deg[1] > 0.5, lax.rsqrt(jnp.maximum(deg[1], 1.0)), 0.0)
    ap = jnp.reshape(
        jnp.stack([_mm(av, p16_ref[rr]) for rr in range(16)], axis=1),
        (PR, 128))
    bp = jnp.reshape(
        jnp.stack([_mm(bv, p16_ref[rr]) for rr in range(16)], axis=1),
        (PR, 128))
    h0a_ref[...] = h0a
    h0b_ref[...] = h0b
    g1a_ref[...] = h0a * ap
    g1b_ref[...] = h0b * ap
    ap_ref[...] = ap
    bp_ref[...] = bp
    cpp_ref[...] = ap * bp


_prep_tc = pl.pallas_call(
    _prep_body,
    out_shape=tuple(jax.ShapeDtypeStruct((PR, 128), jnp.float32)
                    for _ in range(7)),
)


def _round_body(s1_ref, s2_ref, ap_ref, bp_ref, nfa_ref, nfb_ref,
                hia_ref, hib_ref, bdt_ref, bdg1_ref, bdg2_ref,
                bta_ref, btb_ref, b1a_ref, b1b_ref, b2a_ref, b2b_ref,
                nfa2_ref, nfb2_ref, hia2_ref, hib2_ref, g1a_ref, g1b_ref):
    ap = ap_ref[...]
    bp = bp_ref[...]
    f1a = s1_ref[0] * bp
    f1b = s1_ref[1] * bp
    f2a = s2_ref[0] * bp
    f2b = s2_ref[1] * bp
    pieces_t = (nfa_ref[...], nfb_ref[...], f1a, f1b, f2a, f2b)
    ta = jnp.maximum(
        sum(_mm(p, bdt_ref[i, 0]) for i, p in enumerate(pieces_t))
        + bta_ref[...], 0.0)
    tb = jnp.maximum(
        sum(_mm(p, bdt_ref[i, 1]) for i, p in enumerate(pieces_t))
        + btb_ref[...], 0.0)
    pieces_g = (ta, tb, hia_ref[...], hib_ref[...])
    nfa2 = sum(_mm(p, bdg1_ref[i, 0]) for i, p in enumerate(pieces_g)) \
        + b1a_ref[...]
    nfb2 = sum(_mm(p, bdg1_ref[i, 1]) for i, p in enumerate(pieces_g)) \
        + b1b_ref[...]
    hia2 = sum(_mm(p, bdg2_ref[i, 0]) for i, p in enumerate(pieces_g)) \
        + b2a_ref[...]
    hib2 = sum(_mm(p, bdg2_ref[i, 1]) for i, p in enumerate(pieces_g)) \
        + b2b_ref[...]
    nfa2_ref[...] = nfa2
    nfb2_ref[...] = nfb2
    hia2_ref[...] = hia2
    hib2_ref[...] = hib2
    g1a_ref[...] = nfa2 * ap
    g1b_ref[...] = nfb2 * ap


_round_tc = pl.pallas_call(
    _round_body,
    out_shape=tuple(jax.ShapeDtypeStruct((PR, 128), jnp.float32)
                    for _ in range(6)),
)


def _s2s_body(h0a_ref, h0b_ref, nfa_ref, nfb_ref, wih_ref, whh_ref,
              bih_ref, bhh_ref, ws_ref, bs_ref, pa_ref,
              s_ref, st_ref, f_ref, out_ref):
    h0a = h0a_ref[...]
    h0b = h0b_ref[...]
    nfa = nfa_ref[...]
    nfb = nfb_ref[...]
    S = s_ref[...]
    ST = st_ref[...]
    F = f_ref[...]
    wih = wih_ref[...]
    whh = whh_ref[...]
    bih = bih_ref[...][None, :]
    bhh = bhh_ref[...][None, :]
    mask = lax.broadcasted_iota(jnp.int32, (PR, 8), 0) < PRN
    z6 = jnp.zeros((1, HD - HH), jnp.float32)

    def brd(qpiece):
        return jnp.tile(jnp.concatenate([qpiece, z6], 1), (1, 8))

    h = jnp.zeros((1, 2 * H), jnp.float32)
    c = jnp.zeros((1, 2 * H), jnp.float32)
    q_star = jnp.zeros((1, 4 * H), jnp.float32)
    for _ in range(3):
        gates = (lax.dot_general(q_star, wih, (((1,), (1,)), ((), ())),
                                 preferred_element_type=jnp.float32) + bih
                 + lax.dot_general(h, whh, (((1,), (1,)), ((), ())),
                                   preferred_element_type=jnp.float32) + bhh)
        i_g = jax.nn.sigmoid(gates[:, 0:2 * H])
        f_g = jax.nn.sigmoid(gates[:, 2 * H:4 * H])
        g_g = jnp.tanh(gates[:, 4 * H:6 * H])
        o_g = jax.nn.sigmoid(gates[:, 6 * H:8 * H])
        c = f_g * c + i_g * g_g
        h = o_g * jnp.tanh(c)
        eblk = (h0a * brd(h[:, 0:HH]) + h0b * brd(h[:, HH:2 * HH])
                + nfa * brd(h[:, 2 * HH:3 * HH]) + nfb * brd(h[:, 3 * HH:]))
        e8 = jnp.where(mask, _mm(eblk, S), -1e30)
        m = jnp.max(e8)
        p = jnp.where(mask, jnp.exp(e8 - m), 0.0)
        alpha = p / jnp.sum(p)
        w128 = _mm(alpha, ST)
        va = _mm(jnp.sum(w128 * h0a, axis=0, keepdims=True), F)
        vb = _mm(jnp.sum(w128 * h0b, axis=0, keepdims=True), F)
        vna = _mm(jnp.sum(w128 * nfa, axis=0, keepdims=True), F)
        vnb = _mm(jnp.sum(w128 * nfb, axis=0, keepdims=True), F)
        readout = jnp.concatenate(
            [va[:, :HH], vb[:, :HH], vna[:, :HH], vnb[:, :HH]], 1)
        q_star = jnp.concatenate([h, readout], axis=-1)
    y = _mm(q_star, ws_ref[...]) + bs_ref[...][None, :]
    pa = pa_ref[0, 0]
    out_ref[...] = jnp.where(y >= 0, y, pa * y)


_s2s_tc = pl.pallas_call(
    _s2s_body,
    out_shape=jax.ShapeDtypeStruct((1, H), jnp.float32),
)


def kernel(x, edge_attr, edge_index, W_proj, b_proj, W_tag, b_tag,
           W_g1, b_g1, W_g2, b_g2, W_ih, W_hh, b_ih, b_hh, W_s, b_s, prelu_a):
    f32 = jnp.float32
    ei4 = jnp.reshape(edge_index, (2, NS, NBLK2, EB))
    ones1 = jnp.ones((EB,), f32)
    zeros2 = jnp.zeros((2, NPAD), f32)
    zeros16 = jnp.zeros((NPAD, HD), f32)

    ey8 = jnp.eye(8, dtype=f32)
    z6c = jnp.zeros((6,), f32)

    def halfpad(w):
        return jnp.concatenate([w, jnp.zeros((w.shape[0], HD - HH), f32)], 1)

    def bd_batch(ws):
        k = ws.shape[0]
        wp = jnp.pad(ws, ((0, 0), (0, HD - HH), (0, HD - HH)))
        out = (ey8[None, :, None, :, None]
               * wp[:, None, :, None, :]).reshape(k, 128, 128)
        return out

    def bbrd(b):
        return jnp.tile(jnp.concatenate([b, z6c]), 8)[None, :]

    WA = jnp.kron(ey8, halfpad(W_proj[:, :HH]))
    WB = jnp.kron(ey8, halfpad(W_proj[:, HH:]))
    bba = bbrd(b_proj[:HH])
    bbb = bbrd(b_proj[HH:])

    def blocks(w, npc):
        return jnp.reshape(w, (npc, HH, 2, HH)).transpose(0, 2, 1, 3)

    BDT = jnp.reshape(bd_batch(jnp.reshape(blocks(W_tag, 6), (12, HH, HH))),
                      (6, 2, 128, 128))
    BDG1 = jnp.reshape(bd_batch(jnp.reshape(blocks(W_g1, 4), (8, HH, HH))),
                       (4, 2, 128, 128))
    BDG2 = jnp.reshape(bd_batch(jnp.reshape(blocks(W_g2, 4), (8, HH, HH))),
                       (4, 2, 128, 128))
    bta, btb = bbrd(b_tag[:HH]), bbrd(b_tag[HH:])
    b1a, b1b = bbrd(b_g1[:HH]), bbrd(b_g1[HH:])
    b2a, b2b = bbrd(b_g2[:HH]), bbrd(b_g2[HH:])

    ar128 = jnp.arange(128)
    P16 = (ar128[None, :, None]
           == 8 * jnp.arange(16)[:, None, None]
           + (ar128 // HD)[None, None, :]).astype(f32)
    S = (ar128[:, None] // HD == jnp.arange(8)[None, :]).astype(f32)
    ST = S.T
    F = (ar128[:, None] % HD == jnp.arange(HD)[None, :]).astype(f32)

    ei4d = jnp.reshape(edge_index, (2, NW, NHB, EB))
    degp = _degrees_sc(ei4d, ones1, zeros2)
    degp4 = jnp.reshape(degp, (NC, 2, DR, 128))
    x8 = jnp.reshape(x, (PRN, 1024))
    h0a, h0b, g1a, g1b, ap, bp, cpp = _prep_tc(
        x8, WA, WB, bba, bbb, degp4, P16)

    def rs16(t):
        return jnp.reshape(t, (NPAD, HD))

    nfa, nfb, hia, hib = h0a, h0b, h0a, h0b
    for _ in range(3):
        s1, s2, _fb = _round_sc(rs16(g1a), rs16(g1b), ei4, rs16(cpp), zeros16)
        s1r = jnp.reshape(s1, (NC, PR, 128))
        s2r = jnp.reshape(s2, (NC, PR, 128))
        nfa, nfb, hia, hib, g1a, g1b = _round_tc(
            s1r, s2r, ap, bp, nfa, nfb, hia, hib, BDT, BDG1, BDG2,
            bta, btb, b1a, b1b, b2a, b2b)

    return _s2s_tc(h0a, h0b, nfa, nfb, W_ih, W_hh, b_ih, b_hh, W_s, b_s,
                   jnp.reshape(prelu_a, (1, 1)), S, ST, F)


# speedup vs baseline: 1.0030x; 1.0030x over previous
import functools

import jax
import jax.numpy as jnp
from jax import lax
from jax.experimental import pallas as pl
from jax.experimental.pallas import tpu as pltpu
from jax.experimental.pallas import tpu_sc as plsc

N = 10000
NPAD = 10240
E = 320000
H = 20
HH = 10
HD = 16
NC = 2
NS = 16
NW = NC * NS
EPW = E // NW
EPT = E // NS
EB = 1000
NBLK2 = EPT // EB
RPS = NPAD // NS
PR = NPAD // 8
PRN = N // 8
DR = NPAD // 128

_sc_mesh = plsc.VectorSubcoreMesh(core_axis_name="c", subcore_axis_name="s")


NHB = NBLK2 // 2


@functools.partial(
    pl.kernel,
    out_type=jax.ShapeDtypeStruct((NC, 2, NPAD), jnp.float32),
    mesh=_sc_mesh,
    scratch_types=[
        pltpu.VMEM_SHARED((NPAD,), jnp.float32),
        pltpu.VMEM_SHARED((NPAD,), jnp.float32),
        pltpu.VMEM((NHB, EB), jnp.int32),
        pltpu.VMEM((NHB, EB), jnp.int32),
        pltpu.VMEM((EB,), jnp.float32),
    ],
    compiler_params=pltpu.CompilerParams(use_tc_tiling_on_sc=False),
)
def _degrees_sc(ei4_hbm, ones_hbm, zeros_hbm, out_hbm,
                acc_o, acc_i, idx_s, idx_d, ones_v):
    cid = lax.axis_index("c")
    sid = lax.axis_index("s")
    wid = cid * NS + sid

    row0 = sid * RPS
    pltpu.sync_copy(zeros_hbm.at[0, pl.ds(row0, RPS)], acc_o.at[pl.ds(row0, RPS)])
    pltpu.sync_copy(zeros_hbm.at[1, pl.ds(row0, RPS)], acc_i.at[pl.ds(row0, RPS)])
    pltpu.sync_copy(ei4_hbm.at[0, wid], idx_s)
    pltpu.sync_copy(ei4_hbm.at[1, wid], idx_d)
    pltpu.sync_copy(ones_hbm, ones_v)
    plsc.subcore_barrier()

    def body(r, carry):
        pltpu.sync_copy(ones_v, acc_o.at[idx_s.at[r]], add=True)
        pltpu.sync_copy(ones_v, acc_i.at[idx_d.at[r]], add=True)
        return carry
    lax.fori_loop(0, NHB, body, 0)

    plsc.subcore_barrier()
    pltpu.sync_copy(acc_o.at[pl.ds(row0, RPS)], out_hbm.at[cid, 0, pl.ds(row0, RPS)])
    pltpu.sync_copy(acc_i.at[pl.ds(row0, RPS)], out_hbm.at[cid, 1, pl.ds(row0, RPS)])


@functools.partial(
    pl.kernel,
    out_type=(
        jax.ShapeDtypeStruct((NC, NPAD, HD), jnp.float32),
        jax.ShapeDtypeStruct((NC, NPAD, HD), jnp.float32),
    ),
    mesh=_sc_mesh,
    scratch_types=[
        pltpu.VMEM_SHARED((NPAD, HD), jnp.float32),
        pltpu.VMEM_SHARED((NPAD, HD), jnp.float32),
        pltpu.VMEM_SHARED((NPAD, HD), jnp.float32),
        pltpu.VMEM((NBLK2, EB), jnp.int32),
        pltpu.VMEM((NBLK2, EB), jnp.int32),
        pltpu.VMEM((EB, HD), jnp.float32),
        pltpu.VMEM((EB, HD), jnp.float32),
        pltpu.VMEM((RPS, HD), jnp.float32),
        pltpu.VMEM((RPS, HD), jnp.float32),
        pltpu.SemaphoreType.DMA,
        pltpu.SemaphoreType.DMA,
    ],
    compiler_params=pltpu.CompilerParams(use_tc_tiling_on_sc=False),
)
def _round_sc(feat0_hbm, feat1_hbm, ei4_hbm, cbrd_hbm, zeros16_hbm,
              s1_out, s2_out,
              accA, accB, featB, srcv, dstv, rows0, rows1, bufA, bufC,
              sem0, sem1):
    cid = lax.axis_index("c")
    sid = lax.axis_index("s")
    row0 = sid * RPS

    pltpu.sync_copy(zeros16_hbm.at[pl.ds(row0, RPS)], accA.at[pl.ds(row0, RPS)])
    pltpu.sync_copy(zeros16_hbm.at[pl.ds(row0, RPS)], accB.at[pl.ds(row0, RPS)])
    pltpu.sync_copy(ei4_hbm.at[0, sid], srcv)
    pltpu.sync_copy(ei4_hbm.at[1, sid], dstv)
    plsc.subcore_barrier()

    def _edge_pass(gsrc, acc):
        pltpu.async_copy(gsrc.at[srcv.at[0]], rows0, sem0)

        def body(i, carry):
            b0 = 2 * i
            pltpu.async_copy(gsrc.at[srcv.at[b0 + 1]], rows1, sem1)
            pltpu.make_async_copy(gsrc.at[srcv.at[b0]], rows0, sem0).wait()
            pltpu.sync_copy(rows0, acc.at[dstv.at[b0]], add=True)

            @pl.when(i < NBLK2 // 2 - 1)
            def _():
                pltpu.async_copy(gsrc.at[srcv.at[b0 + 2]], rows0, sem0)

            pltpu.make_async_copy(gsrc.at[srcv.at[b0 + 1]], rows1, sem1).wait()
            pltpu.sync_copy(rows1, acc.at[dstv.at[b0 + 1]], add=True)
            return carry
        lax.fori_loop(0, NBLK2 // 2, body, 0)

    @pl.when(cid == 0)
    def _():
        _edge_pass(feat0_hbm, accA)

    @pl.when(cid == 1)
    def _():
        _edge_pass(feat1_hbm, accA)

    plsc.subcore_barrier()
    pltpu.sync_copy(accA.at[pl.ds(row0, RPS)], s1_out.at[cid, pl.ds(row0, RPS)])

    pltpu.sync_copy(accA.at[pl.ds(row0, RPS)], bufA)
    pltpu.sync_copy(cbrd_hbm.at[pl.ds(row0, RPS)], bufC)

    def scale(j, carry):
        bufA[j, :] = bufA[j, :] * bufC[j, :]
        return carry
    lax.fori_loop(0, RPS, scale, 0)
    pltpu.sync_copy(bufA, featB.at[pl.ds(row0, RPS)])
    plsc.subcore_barrier()

    _edge_pass(featB, accB)

    plsc.subcore_barrier()
    pltpu.sync_copy(accB.at[pl.ds(row0, RPS)], s2_out.at[cid, pl.ds(row0, RPS)])


def _mm(a, b):
    return jnp.dot(a, b, preferred_element_type=jnp.float32)


def _prep_body(x8_ref, wa_ref, wb_ref, bba_ref, bbb_ref, degp_ref, p16_ref,
               h0a_ref, h0b_ref, g1a_ref, g1b_ref, ap_ref, bp_ref, cpp_ref):
    x8 = x8_ref[...]
    zr = jnp.zeros((PR - PRN, 128), jnp.float32)
    h0a = jnp.concatenate(
        [jnp.maximum(_mm(x8, wa_ref[...]) + bba_ref[...], 0.0), zr], 0)
    h0b = jnp.concatenate(
        [jnp.maximum(_mm(x8, wb_ref[...]) + bbb_ref[...], 0.0), zr], 0)
    deg = degp_ref[0] + degp_ref[1]
    av = jnp.where(deg[0] > 0.5, lax.rsqrt(jnp.maximum(deg[0], 1.0)), 0.0)
    bv = jnp.where(deg[1] > 0.5, lax.rsqrt(jnp.maximum(deg[1], 1.0)), 0.0)
    ap = jnp.reshape(
        jnp.stack([_mm(av, p16_ref[rr]) for rr in range(16)], axis=1),
        (PR, 128))
    bp = jnp.reshape(
        jnp.stack([_mm(bv, p16_ref[rr]) for rr in range(16)], axis=1),
        (PR, 128))
    h0a_ref[...] = h0a
    h0b_ref[...] = h0b
    g1a_ref[...] = h0a * ap
    g1b_ref[...] = h0b * ap
    ap_ref[...] = ap
    bp_ref[...] = bp
    cpp_ref[...] = ap * bp


_prep_tc = pl.pallas_call(
    _prep_body,
    out_shape=tuple(jax.ShapeDtypeStruct((PR, 128), jnp.float32)
                    for _ in range(7)),
)


def _round_body(s1_ref, s2_ref, ap_ref, bp_ref, nfa_ref, nfb_ref,
                hia_ref, hib_ref, bdt_ref, bdg1_ref, bdg2_ref,
                bta_ref, btb_ref, b1a_ref, b1b_ref, b2a_ref, b2b_ref,
                nfa2_ref, nfb2_ref, hia2_ref, hib2_ref, g1a_ref, g1b_ref):
    ap = ap_ref[...]
    bp = bp_ref[...]
    f1a = s1_ref[0] * bp
    f1b = s1_ref[1] * bp
    f2a = s2_ref[0] * bp
    f2b = s2_ref[1] * bp
    pieces_t = (nfa_ref[...], nfb_ref[...], f1a, f1b, f2a, f2b)
    ta = jnp.maximum(
        sum(_mm(p, bdt_ref[i, 0]) for i, p in enumerate(pieces_t))
        + bta_ref[...], 0.0)
    tb = jnp.maximum(
        sum(_mm(p, bdt_ref[i, 1]) for i, p in enumerate(pieces_t))
        + btb_ref[...], 0.0)
    pieces_g = (ta, tb, hia_ref[...], hib_ref[...])
    nfa2 = sum(_mm(p, bdg1_ref[i, 0]) for i, p in enumerate(pieces_g)) \
        + b1a_ref[...]
    nfb2 = sum(_mm(p, bdg1_ref[i, 1]) for i, p in enumerate(pieces_g)) \
        + b1b_ref[...]
    hia2 = sum(_mm(p, bdg2_ref[i, 0]) for i, p in enumerate(pieces_g)) \
        + b2a_ref[...]
    hib2 = sum(_mm(p, bdg2_ref[i, 1]) for i, p in enumerate(pieces_g)) \
        + b2b_ref[...]
    nfa2_ref[...] = nfa2
    nfb2_ref[...] = nfb2
    hia2_ref[...] = hia2
    hib2_ref[...] = hib2
    g1a_ref[...] = nfa2 * ap
    g1b_ref[...] = nfb2 * ap


_round_tc = pl.pallas_call(
    _round_body,
    out_shape=tuple(jax.ShapeDtypeStruct((PR, 128), jnp.float32)
                    for _ in range(6)),
)


def _s2s_body(h0a_ref, h0b_ref, nfa_ref, nfb_ref, wih_ref, whh_ref,
              bih_ref, bhh_ref, ws_ref, bs_ref, pa_ref,
              s_ref, st_ref, f_ref, out_ref):
    h0a = h0a_ref[...]
    h0b = h0b_ref[...]
    nfa = nfa_ref[...]
    nfb = nfb_ref[...]
    S = s_ref[...]
    ST = st_ref[...]
    F = f_ref[...]
    wih = wih_ref[...]
    whh = whh_ref[...]
    bih = bih_ref[...][None, :]
    bhh = bhh_ref[...][None, :]
    mask = lax.broadcasted_iota(jnp.int32, (PR, 8), 0) < PRN
    z6 = jnp.zeros((1, HD - HH), jnp.float32)

    def brd(qpiece):
        return jnp.tile(jnp.concatenate([qpiece, z6], 1), (1, 8))

    h = jnp.zeros((1, 2 * H), jnp.float32)
    c = jnp.zeros((1, 2 * H), jnp.float32)
    q_star = jnp.zeros((1, 4 * H), jnp.float32)
    for _ in range(3):
        gates = (lax.dot_general(q_star, wih, (((1,), (1,)), ((), ())),
                                 preferred_element_type=jnp.float32) + bih
                 + lax.dot_general(h, whh, (((1,), (1,)), ((), ())),
                                   preferred_element_type=jnp.float32) + bhh)
        i_g = jax.nn.sigmoid(gates[:, 0:2 * H])
        f_g = jax.nn.sigmoid(gates[:, 2 * H:4 * H])
        g_g = jnp.tanh(gates[:, 4 * H:6 * H])
        o_g = jax.nn.sigmoid(gates[:, 6 * H:8 * H])
        c = f_g * c + i_g * g_g
        h = o_g * jnp.tanh(c)
        eblk = (h0a * brd(h[:, 0:HH]) + h0b * brd(h[:, HH:2 * HH])
                + nfa * brd(h[:, 2 * HH:3 * HH]) + nfb * brd(h[:, 3 * HH:]))
        e8 = jnp.where(mask, _mm(eblk, S), -1e30)
        m = jnp.max(e8)
        p = jnp.where(mask, jnp.exp(e8 - m), 0.0)
        alpha = p / jnp.sum(p)
        w128 = _mm(alpha, ST)
        va = _mm(jnp.sum(w128 * h0a, axis=0, keepdims=True), F)
        vb = _mm(jnp.sum(w128 * h0b, axis=0, keepdims=True), F)
        vna = _mm(jnp.sum(w128 * nfa, axis=0, keepdims=True), F)
        vnb = _mm(jnp.sum(w128 * nfb, axis=0, keepdims=True), F)
        readout = jnp.concatenate(
            [va[:, :HH], vb[:, :HH], vna[:, :HH], vnb[:, :HH]], 1)
        q_star = jnp.concatenate([h, readout], axis=-1)
    y = _mm(q_star, ws_ref[...]) + bs_ref[...][None, :]
    pa = pa_ref[0, 0]
    out_ref[...] = jnp.where(y >= 0, y, pa * y)


_s2s_tc = pl.pallas_call(
    _s2s_body,
    out_shape=jax.ShapeDtypeStruct((1, H), jnp.float32),
)


def kernel(x, edge_attr, edge_index, W_proj, b_proj, W_tag, b_tag,
           W_g1, b_g1, W_g2, b_g2, W_ih, W_hh, b_ih, b_hh, W_s, b_s, prelu_a):
    f32 = jnp.float32
    ei4 = jnp.reshape(edge_index, (2, NS, NBLK2, EB))
    ones1 = jnp.ones((EB,), f32)
    zeros2 = jnp.zeros((2, NPAD), f32)
    zeros16 = jnp.zeros((NPAD, HD), f32)

    ey8 = jnp.eye(8, dtype=f32)
    z6c = jnp.zeros((6,), f32)

    def halfpad(w):
        return jnp.concatenate([w, jnp.zeros((w.shape[0], HD - HH), f32)], 1)

    def bd_batch(ws):
        k = ws.shape[0]
        wp = jnp.pad(ws, ((0, 0), (0, HD - HH), (0, HD - HH)))
        out = (ey8[None, :, None, :, None]
               * wp[:, None, :, None, :]).reshape(k, 128, 128)
        return out

    def bbrd(b):
        return jnp.tile(jnp.concatenate([b, z6c]), 8)[None, :]

    WA = jnp.kron(ey8, halfpad(W_proj[:, :HH]))
    WB = jnp.kron(ey8, halfpad(W_proj[:, HH:]))
    bba = bbrd(b_proj[:HH])
    bbb = bbrd(b_proj[HH:])

    def blocks(w, npc):
        return jnp.reshape(w, (npc, HH, 2, HH)).transpose(0, 2, 1, 3)

    BDT = jnp.reshape(bd_batch(jnp.reshape(blocks(W_tag, 6), (12, HH, HH))),
                      (6, 2, 128, 128))
    BDG1 = jnp.reshape(bd_batch(jnp.reshape(blocks(W_g1, 4), (8, HH, HH))),
                       (4, 2, 128, 128))
    BDG2 = jnp.reshape(bd_batch(jnp.reshape(blocks(W_g2, 4), (8, HH, HH))),
                       (4, 2, 128, 128))
    bta, btb = bbrd(b_tag[:HH]), bbrd(b_tag[HH:])
    b1a, b1b = bbrd(b_g1[:HH]), bbrd(b_g1[HH:])
    b2a, b2b = bbrd(b_g2[:HH]), bbrd(b_g2[HH:])

    ar128 = jnp.arange(128)
    P16 = (ar128[None, :, None]
           == 8 * jnp.arange(16)[:, None, None]
           + (ar128 // HD)[None, None, :]).astype(f32)
    S = (ar128[:, None] // HD == jnp.arange(8)[None, :]).astype(f32)
    ST = S.T
    F = (ar128[:, None] % HD == jnp.arange(HD)[None, :]).astype(f32)

    ei4d = jnp.reshape(edge_index, (2, NW, NHB, EB))
    degp = _degrees_sc(ei4d, ones1, zeros2)
    degp4 = jnp.reshape(degp, (NC, 2, DR, 128))
    x8 = jnp.reshape(x, (PRN, 1024))
    h0a, h0b, g1a, g1b, ap, bp, cpp = _prep_tc(
        x8, WA, WB, bba, bbb, degp4, P16)

    def rs16(t):
        return jnp.reshape(t, (NPAD, HD))

    nfa, nfb, hia, hib = h0a, h0b, h0a, h0b
    for _ in range(3):
        s1, s2 = _round_sc(rs16(g1a), rs16(g1b), ei4, rs16(cpp), zeros16)
        s1r = jnp.reshape(s1, (NC, PR, 128))
        s2r = jnp.reshape(s2, (NC, PR, 128))
        nfa, nfb, hia, hib, g1a, g1b = _round_tc(
            s1r, s2r, ap, bp, nfa, nfb, hia, hib, BDT, BDG1, BDG2,
            bta, btb, b1a, b1b, b2a, b2b)

    return _s2s_tc(h0a, h0b, nfa, nfb, W_ih, W_hh, b_ih, b_hh, W_s, b_s,
                   jnp.reshape(prelu_a, (1, 1)), S, ST, F)


# speedup vs baseline: 1.0162x; 1.0132x over previous
import functools

import jax
import jax.numpy as jnp
from jax import lax
from jax.experimental import pallas as pl
from jax.experimental.pallas import tpu as pltpu
from jax.experimental.pallas import tpu_sc as plsc

N = 10000
NPAD = 10240
E = 320000
H = 20
HH = 10
HD = 16
NC = 2
NS = 16
NW = NC * NS
EPW = E // NW
EPT = E // NS
EB = 1000
NBLK2 = EPT // EB
RPS = NPAD // NS
PR = NPAD // 8
PRN = N // 8
DR = NPAD // 128

_sc_mesh = plsc.VectorSubcoreMesh(core_axis_name="c", subcore_axis_name="s")


NHB = NBLK2 // 2


@functools.partial(
    pl.kernel,
    out_type=jax.ShapeDtypeStruct((NC, 2, NPAD), jnp.float32),
    mesh=_sc_mesh,
    scratch_types=[
        pltpu.VMEM_SHARED((NPAD,), jnp.float32),
        pltpu.VMEM_SHARED((NPAD,), jnp.float32),
        pltpu.VMEM((NHB, EB), jnp.int32),
        pltpu.VMEM((NHB, EB), jnp.int32),
        pltpu.VMEM((EB,), jnp.float32),
    ],
    compiler_params=pltpu.CompilerParams(use_tc_tiling_on_sc=False),
)
def _degrees_sc(ei4_hbm, ones_hbm, zeros_hbm, out_hbm,
                acc_o, acc_i, idx_s, idx_d, ones_v):
    cid = lax.axis_index("c")
    sid = lax.axis_index("s")

    row0 = sid * RPS
    pltpu.sync_copy(zeros_hbm.at[0, pl.ds(row0, RPS)], acc_o.at[pl.ds(row0, RPS)])
    pltpu.sync_copy(zeros_hbm.at[1, pl.ds(row0, RPS)], acc_i.at[pl.ds(row0, RPS)])
    pltpu.sync_copy(ei4_hbm.at[0, sid, pl.ds(cid * NHB, NHB)], idx_s)
    pltpu.sync_copy(ei4_hbm.at[1, sid, pl.ds(cid * NHB, NHB)], idx_d)
    pltpu.sync_copy(ones_hbm, ones_v)
    plsc.subcore_barrier()

    def body(r, carry):
        pltpu.sync_copy(ones_v, acc_o.at[idx_s.at[r]], add=True)
        pltpu.sync_copy(ones_v, acc_i.at[idx_d.at[r]], add=True)
        return carry
    lax.fori_loop(0, NHB, body, 0)

    plsc.subcore_barrier()
    pltpu.sync_copy(acc_o.at[pl.ds(row0, RPS)], out_hbm.at[cid, 0, pl.ds(row0, RPS)])
    pltpu.sync_copy(acc_i.at[pl.ds(row0, RPS)], out_hbm.at[cid, 1, pl.ds(row0, RPS)])


@functools.partial(
    pl.kernel,
    out_type=(
        jax.ShapeDtypeStruct((NC, NPAD, HD), jnp.float32),
        jax.ShapeDtypeStruct((NC, NPAD, HD), jnp.float32),
    ),
    mesh=_sc_mesh,
    scratch_types=[
        pltpu.VMEM_SHARED((NPAD, HD), jnp.float32),
        pltpu.VMEM_SHARED((NPAD, HD), jnp.float32),
        pltpu.VMEM_SHARED((NPAD, HD), jnp.float32),
        pltpu.VMEM((NBLK2, EB), jnp.int32),
        pltpu.VMEM((NBLK2, EB), jnp.int32),
        pltpu.VMEM((EB, HD), jnp.float32),
        pltpu.VMEM((EB, HD), jnp.float32),
        pltpu.VMEM((RPS, HD), jnp.float32),
        pltpu.VMEM((RPS, HD), jnp.float32),
        pltpu.SemaphoreType.DMA,
        pltpu.SemaphoreType.DMA,
    ],
    compiler_params=pltpu.CompilerParams(use_tc_tiling_on_sc=False),
)
def _round_sc(feat0_hbm, feat1_hbm, ei4_hbm, cbrd_hbm, zeros16_hbm,
              s1_out, s2_out,
              accA, accB, featB, srcv, dstv, rows0, rows1, bufA, bufC,
              sem0, sem1):
    cid = lax.axis_index("c")
    sid = lax.axis_index("s")
    row0 = sid * RPS

    pltpu.sync_copy(zeros16_hbm.at[pl.ds(row0, RPS)], accA.at[pl.ds(row0, RPS)])
    pltpu.sync_copy(zeros16_hbm.at[pl.ds(row0, RPS)], accB.at[pl.ds(row0, RPS)])
    pltpu.sync_copy(ei4_hbm.at[0, sid], srcv)
    pltpu.sync_copy(ei4_hbm.at[1, sid], dstv)
    plsc.subcore_barrier()

    def _edge_pass(gsrc, acc):
        pltpu.async_copy(gsrc.at[srcv.at[0]], rows0, sem0)

        def body(i, carry):
            b0 = 2 * i
            pltpu.async_copy(gsrc.at[srcv.at[b0 + 1]], rows1, sem1)
            pltpu.make_async_copy(gsrc.at[srcv.at[b0]], rows0, sem0).wait()
            pltpu.sync_copy(rows0, acc.at[dstv.at[b0]], add=True)

            @pl.when(i < NBLK2 // 2 - 1)
            def _():
                pltpu.async_copy(gsrc.at[srcv.at[b0 + 2]], rows0, sem0)

            pltpu.make_async_copy(gsrc.at[srcv.at[b0 + 1]], rows1, sem1).wait()
            pltpu.sync_copy(rows1, acc.at[dstv.at[b0 + 1]], add=True)
            return carry
        lax.fori_loop(0, NBLK2 // 2, body, 0)

    @pl.when(cid == 0)
    def _():
        _edge_pass(feat0_hbm, accA)

    @pl.when(cid == 1)
    def _():
        _edge_pass(feat1_hbm, accA)

    plsc.subcore_barrier()
    pltpu.sync_copy(accA.at[pl.ds(row0, RPS)], s1_out.at[cid, pl.ds(row0, RPS)])

    pltpu.sync_copy(accA.at[pl.ds(row0, RPS)], bufA)
    pltpu.sync_copy(cbrd_hbm.at[pl.ds(row0, RPS)], bufC)

    def scale(j, carry):
        bufA[j, :] = bufA[j, :] * bufC[j, :]
        return carry
    lax.fori_loop(0, RPS, scale, 0)
    pltpu.sync_copy(bufA, featB.at[pl.ds(row0, RPS)])
    plsc.subcore_barrier()

    _edge_pass(featB, accB)

    plsc.subcore_barrier()
    pltpu.sync_copy(accB.at[pl.ds(row0, RPS)], s2_out.at[cid, pl.ds(row0, RPS)])


def _mm(a, b):
    return jnp.dot(a, b, preferred_element_type=jnp.float32)


def _prep_body(x8_ref, wa_ref, wb_ref, bba_ref, bbb_ref, degp_ref, p16_ref,
               h0a_ref, h0b_ref, g1a_ref, g1b_ref, ap_ref, bp_ref, cpp_ref):
    x8 = x8_ref[...]
    zr = jnp.zeros((PR - PRN, 128), jnp.float32)
    h0a = jnp.concatenate(
        [jnp.maximum(_mm(x8, wa_ref[...]) + bba_ref[...], 0.0), zr], 0)
    h0b = jnp.concatenate(
        [jnp.maximum(_mm(x8, wb_ref[...]) + bbb_ref[...], 0.0), zr], 0)
    deg = degp_ref[0] + degp_ref[1]
    av = jnp.where(deg[0] > 0.5, lax.rsqrt(jnp.maximum(deg[0], 1.0)), 0.0)
    bv = jnp.where(deg[1] > 0.5, lax.rsqrt(jnp.maximum(deg[1], 1.0)), 0.0)
    ap = jnp.reshape(
        jnp.stack([_mm(av, p16_ref[rr]) for rr in range(16)], axis=1),
        (PR, 128))
    bp = jnp.reshape(
        jnp.stack([_mm(bv, p16_ref[rr]) for rr in range(16)], axis=1),
        (PR, 128))
    h0a_ref[...] = h0a
    h0b_ref[...] = h0b
    g1a_ref[...] = h0a * ap
    g1b_ref[...] = h0b * ap
    ap_ref[...] = ap
    bp_ref[...] = bp
    cpp_ref[...] = ap * bp


_prep_tc = pl.pallas_call(
    _prep_body,
    out_shape=tuple(jax.ShapeDtypeStruct((PR, 128), jnp.float32)
                    for _ in range(7)),
)


def _round_body(s1_ref, s2_ref, ap_ref, bp_ref, nfa_ref, nfb_ref,
                hia_ref, hib_ref, bdt_ref, bdg1_ref, bdg2_ref,
                bta_ref, btb_ref, b1a_ref, b1b_ref, b2a_ref, b2b_ref,
                nfa2_ref, nfb2_ref, hia2_ref, hib2_ref, g1a_ref, g1b_ref):
    ap = ap_ref[...]
    bp = bp_ref[...]
    f1a = s1_ref[0] * bp
    f1b = s1_ref[1] * bp
    f2a = s2_ref[0] * bp
    f2b = s2_ref[1] * bp
    pieces_t = (nfa_ref[...], nfb_ref[...], f1a, f1b, f2a, f2b)
    ta = jnp.maximum(
        sum(_mm(p, bdt_ref[i, 0]) for i, p in enumerate(pieces_t))
        + bta_ref[...], 0.0)
    tb = jnp.maximum(
        sum(_mm(p, bdt_ref[i, 1]) for i, p in enumerate(pieces_t))
        + btb_ref[...], 0.0)
    pieces_g = (ta, tb, hia_ref[...], hib_ref[...])
    nfa2 = sum(_mm(p, bdg1_ref[i, 0]) for i, p in enumerate(pieces_g)) \
        + b1a_ref[...]
    nfb2 = sum(_mm(p, bdg1_ref[i, 1]) for i, p in enumerate(pieces_g)) \
        + b1b_ref[...]
    hia2 = sum(_mm(p, bdg2_ref[i, 0]) for i, p in enumerate(pieces_g)) \
        + b2a_ref[...]
    hib2 = sum(_mm(p, bdg2_ref[i, 1]) for i, p in enumerate(pieces_g)) \
        + b2b_ref[...]
    nfa2_ref[...] = nfa2
    nfb2_ref[...] = nfb2
    hia2_ref[...] = hia2
    hib2_ref[...] = hib2
    g1a_ref[...] = nfa2 * ap
    g1b_ref[...] = nfb2 * ap


_round_tc = pl.pallas_call(
    _round_body,
    out_shape=tuple(jax.ShapeDtypeStruct((PR, 128), jnp.float32)
                    for _ in range(6)),
)


def _s2s_body(s1_ref, s2_ref, bp_ref, nfa_ref, nfb_ref, hia_ref, hib_ref,
              bdt_ref, bdg1_ref, bta_ref, btb_ref, b1a_ref, b1b_ref,
              h0a_ref, h0b_ref, wih_ref, whh_ref,
              bih_ref, bhh_ref, ws_ref, bs_ref, pa_ref,
              s_ref, st_ref, f_ref, out_ref):
    bp = bp_ref[...]
    f1a = s1_ref[0] * bp
    f1b = s1_ref[1] * bp
    f2a = s2_ref[0] * bp
    f2b = s2_ref[1] * bp
    pieces_t = (nfa_ref[...], nfb_ref[...], f1a, f1b, f2a, f2b)
    ta = jnp.maximum(
        sum(_mm(p, bdt_ref[i, 0]) for i, p in enumerate(pieces_t))
        + bta_ref[...], 0.0)
    tb = jnp.maximum(
        sum(_mm(p, bdt_ref[i, 1]) for i, p in enumerate(pieces_t))
        + btb_ref[...], 0.0)
    pieces_g = (ta, tb, hia_ref[...], hib_ref[...])
    nfa = sum(_mm(p, bdg1_ref[i, 0]) for i, p in enumerate(pieces_g)) \
        + b1a_ref[...]
    nfb = sum(_mm(p, bdg1_ref[i, 1]) for i, p in enumerate(pieces_g)) \
        + b1b_ref[...]
    h0a = h0a_ref[...]
    h0b = h0b_ref[...]
    S = s_ref[...]
    ST = st_ref[...]
    F = f_ref[...]
    wih = wih_ref[...]
    whh = whh_ref[...]
    bih = bih_ref[...][None, :]
    bhh = bhh_ref[...][None, :]
    mask = lax.broadcasted_iota(jnp.int32, (PR, 8), 0) < PRN
    z6 = jnp.zeros((1, HD - HH), jnp.float32)

    def brd(qpiece):
        return jnp.tile(jnp.concatenate([qpiece, z6], 1), (1, 8))

    h = jnp.zeros((1, 2 * H), jnp.float32)
    c = jnp.zeros((1, 2 * H), jnp.float32)
    q_star = jnp.zeros((1, 4 * H), jnp.float32)
    for _ in range(3):
        gates = (lax.dot_general(q_star, wih, (((1,), (1,)), ((), ())),
                                 preferred_element_type=jnp.float32) + bih
                 + lax.dot_general(h, whh, (((1,), (1,)), ((), ())),
                                   preferred_element_type=jnp.float32) + bhh)
        i_g = jax.nn.sigmoid(gates[:, 0:2 * H])
        f_g = jax.nn.sigmoid(gates[:, 2 * H:4 * H])
        g_g = jnp.tanh(gates[:, 4 * H:6 * H])
        o_g = jax.nn.sigmoid(gates[:, 6 * H:8 * H])
        c = f_g * c + i_g * g_g
        h = o_g * jnp.tanh(c)
        eblk = (h0a * brd(h[:, 0:HH]) + h0b * brd(h[:, HH:2 * HH])
                + nfa * brd(h[:, 2 * HH:3 * HH]) + nfb * brd(h[:, 3 * HH:]))
        e8 = jnp.where(mask, _mm(eblk, S), -1e30)
        m = jnp.max(e8)
        p = jnp.where(mask, jnp.exp(e8 - m), 0.0)
        alpha = p / jnp.sum(p)
        w128 = _mm(alpha, ST)
        va = _mm(jnp.sum(w128 * h0a, axis=0, keepdims=True), F)
        vb = _mm(jnp.sum(w128 * h0b, axis=0, keepdims=True), F)
        vna = _mm(jnp.sum(w128 * nfa, axis=0, keepdims=True), F)
        vnb = _mm(jnp.sum(w128 * nfb, axis=0, keepdims=True), F)
        readout = jnp.concatenate(
            [va[:, :HH], vb[:, :HH], vna[:, :HH], vnb[:, :HH]], 1)
        q_star = jnp.concatenate([h, readout], axis=-1)
    y = _mm(q_star, ws_ref[...]) + bs_ref[...][None, :]
    pa = pa_ref[0, 0]
    out_ref[...] = jnp.where(y >= 0, y, pa * y)


_s2s_tc = pl.pallas_call(
    _s2s_body,
    out_shape=jax.ShapeDtypeStruct((1, H), jnp.float32),
)


def kernel(x, edge_attr, edge_index, W_proj, b_proj, W_tag, b_tag,
           W_g1, b_g1, W_g2, b_g2, W_ih, W_hh, b_ih, b_hh, W_s, b_s, prelu_a):
    f32 = jnp.float32
    ei4 = jnp.reshape(edge_index, (2, NS, NBLK2, EB))
    ones1 = jnp.ones((EB,), f32)
    zeros2 = jnp.zeros((2, NPAD), f32)
    zeros16 = jnp.zeros((NPAD, HD), f32)

    ey8 = jnp.eye(8, dtype=f32)
    z6c = jnp.zeros((6,), f32)

    def halfpad(w):
        return jnp.concatenate([w, jnp.zeros((w.shape[0], HD - HH), f32)], 1)

    def bd_batch(ws):
        k = ws.shape[0]
        wp = jnp.pad(ws, ((0, 0), (0, HD - HH), (0, HD - HH)))
        out = (ey8[None, :, None, :, None]
               * wp[:, None, :, None, :]).reshape(k, 128, 128)
        return out

    def bbrd(b):
        return jnp.tile(jnp.concatenate([b, z6c]), 8)[None, :]

    WA = jnp.kron(ey8, halfpad(W_proj[:, :HH]))
    WB = jnp.kron(ey8, halfpad(W_proj[:, HH:]))
    bba = bbrd(b_proj[:HH])
    bbb = bbrd(b_proj[HH:])

    def blocks(w, npc):
        return jnp.reshape(w, (npc, HH, 2, HH)).transpose(0, 2, 1, 3)

    BDT = jnp.reshape(bd_batch(jnp.reshape(blocks(W_tag, 6), (12, HH, HH))),
                      (6, 2, 128, 128))
    BDG1 = jnp.reshape(bd_batch(jnp.reshape(blocks(W_g1, 4), (8, HH, HH))),
                       (4, 2, 128, 128))
    BDG2 = jnp.reshape(bd_batch(jnp.reshape(blocks(W_g2, 4), (8, HH, HH))),
                       (4, 2, 128, 128))
    bta, btb = bbrd(b_tag[:HH]), bbrd(b_tag[HH:])
    b1a, b1b = bbrd(b_g1[:HH]), bbrd(b_g1[HH:])
    b2a, b2b = bbrd(b_g2[:HH]), bbrd(b_g2[HH:])

    ar128 = jnp.arange(128)
    P16 = (ar128[None, :, None]
           == 8 * jnp.arange(16)[:, None, None]
           + (ar128 // HD)[None, None, :]).astype(f32)
    S = (ar128[:, None] // HD == jnp.arange(8)[None, :]).astype(f32)
    ST = S.T
    F = (ar128[:, None] % HD == jnp.arange(HD)[None, :]).astype(f32)

    degp = _degrees_sc(ei4, ones1, zeros2)
    degp4 = jnp.reshape(degp, (NC, 2, DR, 128))
    x8 = jnp.reshape(x, (PRN, 1024))
    h0a, h0b, g1a, g1b, ap, bp, cpp = _prep_tc(
        x8, WA, WB, bba, bbb, degp4, P16)

    def rs16(t):
        return jnp.reshape(t, (NPAD, HD))

    nfa, nfb, hia, hib = h0a, h0b, h0a, h0b
    for _ in range(2):
        s1, s2 = _round_sc(rs16(g1a), rs16(g1b), ei4, rs16(cpp), zeros16)
        s1r = jnp.reshape(s1, (NC, PR, 128))
        s2r = jnp.reshape(s2, (NC, PR, 128))
        nfa, nfb, hia, hib, g1a, g1b = _round_tc(
            s1r, s2r, ap, bp, nfa, nfb, hia, hib, BDT, BDG1, BDG2,
            bta, btb, b1a, b1b, b2a, b2b)

    s1, s2 = _round_sc(rs16(g1a), rs16(g1b), ei4, rs16(cpp), zeros16)
    s1r = jnp.reshape(s1, (NC, PR, 128))
    s2r = jnp.reshape(s2, (NC, PR, 128))
    return _s2s_tc(s1r, s2r, bp, nfa, nfb, hia, hib, BDT, BDG1,
                   bta, btb, b1a, b1b, h0a, h0b,
                   W_ih, W_hh, b_ih, b_hh, W_s, b_s,
                   jnp.reshape(prelu_a, (1, 1)), S, ST, F)
